# baseline (speedup 1.0000x reference)
import jax
import jax.numpy as jnp
from jax import lax
from jax.experimental import pallas as pl
from jax.experimental.pallas import tpu as pltpu

_T = 16384
_D = 4096
_E = 64
_K = 8
_BT = 256
_GRID = _T // _BT


def _router_body(x_ref, wt_ref, b_ref, idx_ref, prob_ref, aux_ref,
                 psum_ref, fsum_ref):
    step = pl.program_id(0)

    @pl.when(step == 0)
    def _init():
        psum_ref[...] = jnp.zeros_like(psum_ref)
        fsum_ref[...] = jnp.zeros_like(fsum_ref)

    logits = jnp.dot(x_ref[...], wt_ref[...],
                     preferred_element_type=jnp.float32)
    logits = logits + b_ref[...]

    m = jnp.max(logits, axis=-1, keepdims=True)
    e = jnp.exp(logits - m)
    s = jnp.sum(e, axis=-1, keepdims=True)
    probs = e / s

    iota = lax.broadcasted_iota(jnp.int32, (_BT, _E), 1)
    p = probs
    vals = []
    idxs = []
    onehot_acc = jnp.zeros((_BT, _E), jnp.float32)
    neg_inf = jnp.float32(-jnp.inf)
    for _ in range(_K):
        mx = jnp.max(p, axis=-1, keepdims=True)
        sel = jnp.min(jnp.where(p == mx, iota, _E), axis=-1, keepdims=True)
        hit = iota == sel
        onehot_acc += hit.astype(jnp.float32)
        vals.append(mx)
        idxs.append(sel)
        p = jnp.where(hit, neg_inf, p)

    topv = jnp.concatenate(vals, axis=1)
    topi = jnp.concatenate(idxs, axis=1)
    denom = jnp.sum(topv, axis=1, keepdims=True) + jnp.float32(1e-9)
    prob_ref[...] = topv / denom
    idx_ref[...] = topi

    psum_ref[...] += jnp.sum(probs, axis=0, keepdims=True)
    fsum_ref[...] += jnp.sum(onehot_acc, axis=0, keepdims=True)

    @pl.when(step == _GRID - 1)
    def _finish():
        scale = jnp.float32(float(_E) / (float(_T) * float(_T) * float(_K)))
        aux_ref[...] = scale * jnp.sum(psum_ref[...] * fsum_ref[...],
                                       axis=1, keepdims=True)


def kernel(x, W, b):
    wt = W.T
    b2 = b.reshape(1, _E)
    idx, prob, aux = pl.pallas_call(
        _router_body,
        grid=(_GRID,),
        in_specs=[
            pl.BlockSpec((_BT, _D), lambda i: (i, 0)),
            pl.BlockSpec((_D, _E), lambda i: (0, 0)),
            pl.BlockSpec((1, _E), lambda i: (0, 0)),
        ],
        out_specs=[
            pl.BlockSpec((_BT, _K), lambda i: (i, 0)),
            pl.BlockSpec((_BT, _K), lambda i: (i, 0)),
            pl.BlockSpec((1, 1), lambda i: (0, 0)),
        ],
        out_shape=[
            jax.ShapeDtypeStruct((_T, _K), jnp.int32),
            jax.ShapeDtypeStruct((_T, _K), jnp.float32),
            jax.ShapeDtypeStruct((1, 1), jnp.float32),
        ],
        scratch_shapes=[
            pltpu.VMEM((1, _E), jnp.float32),
            pltpu.VMEM((1, _E), jnp.float32),
        ],
        compiler_params=pltpu.CompilerParams(
            dimension_semantics=("arbitrary",),
        ),
    )(x, wt, b2)
    return idx, prob, aux[0, 0]


# speedup vs baseline: 1.5627x; 1.5627x over previous
import jax
import jax.numpy as jnp
from jax import lax
from jax.experimental import pallas as pl
from jax.experimental.pallas import tpu as pltpu

_T = 16384
_D = 4096
_E = 64
_K = 8
_BT = 256
_GRID = _T // _BT


def _router_body(w_ref, x_ref, b_ref, idx_ref, prob_ref, aux_ref,
                 psum_ref, fsum_ref):
    step = pl.program_id(0)

    @pl.when(step == 0)
    def _init():
        psum_ref[...] = jnp.zeros_like(psum_ref)
        fsum_ref[...] = jnp.zeros_like(fsum_ref)

    logits_t = lax.dot_general(
        w_ref[...], x_ref[...],
        dimension_numbers=(((1,), (1,)), ((), ())),
        preferred_element_type=jnp.float32)
    logits_t = logits_t + b_ref[...]

    row = lax.broadcasted_iota(jnp.int32, (_E, _BT), 0)
    bits = lax.bitcast_convert_type(logits_t, jnp.int32)
    key = lax.bitcast_convert_type((bits & ~63) | (63 - row), jnp.float32)

    neg_inf = jnp.float32(-jnp.inf)
    rows = []
    for _ in range(_K):
        mx = jnp.max(key, axis=0, keepdims=True)
        rows.append(mx)
        key = jnp.where(key == mx, neg_inf, key)

    keys8 = jnp.concatenate(rows, axis=0)

    m0 = rows[0]
    e_t = jnp.exp(logits_t - m0)
    zinv = jnp.float32(1.0) / jnp.sum(e_t, axis=0, keepdims=True)
    probs_t = e_t * zinv
    psum_ref[...] += jnp.sum(probs_t, axis=1, keepdims=True)

    sel = (key == neg_inf).astype(jnp.float32)
    fsum_ref[...] += jnp.sum(sel, axis=1, keepdims=True)

    p8 = jnp.exp(keys8 - m0) * zinv
    s8 = jnp.sum(p8, axis=0, keepdims=True)
    out_p = p8 / (s8 + jnp.float32(1e-9))
    idx8 = 63 - (lax.bitcast_convert_type(keys8, jnp.int32) & 63)

    prob_ref[...] = out_p.T
    idx_ref[...] = idx8.T

    @pl.when(step == _GRID - 1)
    def _finish():
        scale = jnp.float32(float(_E) / (float(_T) * float(_T) * float(_K)))
        aux_ref[...] = scale * jnp.sum(psum_ref[...] * fsum_ref[...],
                                       axis=0, keepdims=True)


def kernel(x, W, b):
    b2 = b.reshape(_E, 1)
    idx, prob, aux = pl.pallas_call(
        _router_body,
        grid=(_GRID,),
        in_specs=[
            pl.BlockSpec((_E, _D), lambda i: (0, 0)),
            pl.BlockSpec((_BT, _D), lambda i: (i, 0)),
            pl.BlockSpec((_E, 1), lambda i: (0, 0)),
        ],
        out_specs=[
            pl.BlockSpec((_BT, _K), lambda i: (i, 0)),
            pl.BlockSpec((_BT, _K), lambda i: (i, 0)),
            pl.BlockSpec((1, 1), lambda i: (0, 0)),
        ],
        out_shape=[
            jax.ShapeDtypeStruct((_T, _K), jnp.int32),
            jax.ShapeDtypeStruct((_T, _K), jnp.float32),
            jax.ShapeDtypeStruct((1, 1), jnp.float32),
        ],
        scratch_shapes=[
            pltpu.VMEM((_E, 1), jnp.float32),
            pltpu.VMEM((_E, 1), jnp.float32),
        ],
        compiler_params=pltpu.CompilerParams(
            dimension_semantics=("arbitrary",),
        ),
    )(W, x, b2)
    return idx, prob, aux[0, 0]


# speedup vs baseline: 1.8536x; 1.1862x over previous
import jax
import jax.numpy as jnp
from jax import lax
from jax.experimental import pallas as pl
from jax.experimental.pallas import tpu as pltpu

_T = 16384
_D = 4096
_E = 64
_K = 8
_BT = 512
_GRID = _T // _BT


def _router_body(w_ref, x_ref, b_ref, idx_ref, prob_ref, aux_ref,
                 psum_ref, fsum_ref):
    step = pl.program_id(0)

    @pl.when(step == 0)
    def _init():
        psum_ref[...] = jnp.zeros_like(psum_ref)
        fsum_ref[...] = jnp.zeros_like(fsum_ref)

    logits_t = lax.dot_general(
        w_ref[...], x_ref[...],
        dimension_numbers=(((1,), (1,)), ((), ())),
        preferred_element_type=jnp.float32)
    logits_t = logits_t + b_ref[...]

    row = lax.broadcasted_iota(jnp.int32, (_E, _BT), 0)
    bits = lax.bitcast_convert_type(logits_t, jnp.int32)
    key = lax.bitcast_convert_type((bits & ~63) | (63 - row), jnp.float32)

    neg_inf = jnp.float32(-jnp.inf)
    rows = []
    for _ in range(_K):
        mx = jnp.max(key, axis=0, keepdims=True)
        rows.append(mx)
        key = jnp.where(key == mx, neg_inf, key)

    keys8 = jnp.concatenate(rows, axis=0)

    m0 = rows[0]
    e_t = jnp.exp(logits_t - m0)
    zinv = jnp.float32(1.0) / jnp.sum(e_t, axis=0, keepdims=True)
    probs_t = e_t * zinv
    psum_ref[...] += jnp.sum(probs_t, axis=1, keepdims=True)

    sel = (key == neg_inf).astype(jnp.float32)
    fsum_ref[...] += jnp.sum(sel, axis=1, keepdims=True)

    p8 = jnp.exp(keys8 - m0) * zinv
    s8 = jnp.sum(p8, axis=0, keepdims=True)
    out_p = p8 / (s8 + jnp.float32(1e-9))
    idx8 = 63 - (lax.bitcast_convert_type(keys8, jnp.int32) & 63)

    prob_ref[...] = out_p.T
    idx_ref[...] = idx8.T

    @pl.when(step == _GRID - 1)
    def _finish():
        scale = jnp.float32(float(_E) / (float(_T) * float(_T) * float(_K)))
        aux_ref[...] = scale * jnp.sum(psum_ref[...] * fsum_ref[...],
                                       axis=0, keepdims=True)


def kernel(x, W, b):
    b2 = b.reshape(_E, 1)
    idx, prob, aux = pl.pallas_call(
        _router_body,
        grid=(_GRID,),
        in_specs=[
            pl.BlockSpec((_E, _D), lambda i: (0, 0)),
            pl.BlockSpec((_BT, _D), lambda i: (i, 0)),
            pl.BlockSpec((_E, 1), lambda i: (0, 0)),
        ],
        out_specs=[
            pl.BlockSpec((_BT, _K), lambda i: (i, 0)),
            pl.BlockSpec((_BT, _K), lambda i: (i, 0)),
            pl.BlockSpec((1, 1), lambda i: (0, 0)),
        ],
        out_shape=[
            jax.ShapeDtypeStruct((_T, _K), jnp.int32),
            jax.ShapeDtypeStruct((_T, _K), jnp.float32),
            jax.ShapeDtypeStruct((1, 1), jnp.float32),
        ],
        scratch_shapes=[
            pltpu.VMEM((_E, 1), jnp.float32),
            pltpu.VMEM((_E, 1), jnp.float32),
        ],
        compiler_params=pltpu.CompilerParams(
            dimension_semantics=("arbitrary",),
        ),
    )(W, x, b2)
    return idx, prob, aux[0, 0]


# speedup vs baseline: 1.9555x; 1.0550x over previous
import jax
import jax.numpy as jnp
from jax import lax
from jax.experimental import pallas as pl
from jax.experimental.pallas import tpu as pltpu

_T = 16384
_D = 4096
_E = 64
_K = 8
_BT = 1024
_GRID = _T // _BT


def _router_body(w_ref, x_ref, b_ref, idx_ref, prob_ref, aux_ref,
                 psum_ref, fsum_ref):
    step = pl.program_id(0)

    @pl.when(step == 0)
    def _init():
        psum_ref[...] = jnp.zeros_like(psum_ref)
        fsum_ref[...] = jnp.zeros_like(fsum_ref)

    logits_t = lax.dot_general(
        w_ref[...], x_ref[...],
        dimension_numbers=(((1,), (1,)), ((), ())),
        preferred_element_type=jnp.float32)
    logits_t = logits_t + b_ref[...]

    row = lax.broadcasted_iota(jnp.int32, (_E, _BT), 0)
    bits = lax.bitcast_convert_type(logits_t, jnp.int32)
    key = lax.bitcast_convert_type((bits & ~63) | (63 - row), jnp.float32)

    neg_inf = jnp.float32(-jnp.inf)
    rows = []
    for _ in range(_K):
        mx = jnp.max(key, axis=0, keepdims=True)
        rows.append(mx)
        key = jnp.where(key == mx, neg_inf, key)

    keys8 = jnp.concatenate(rows, axis=0)

    m0 = rows[0]
    e_t = jnp.exp(logits_t - m0)
    zinv = jnp.float32(1.0) / jnp.sum(e_t, axis=0, keepdims=True)
    probs_t = e_t * zinv
    psum_ref[...] += jnp.sum(probs_t, axis=1, keepdims=True)

    sel = (key == neg_inf).astype(jnp.float32)
    fsum_ref[...] += jnp.sum(sel, axis=1, keepdims=True)

    p8 = jnp.exp(keys8 - m0) * zinv
    s8 = jnp.sum(p8, axis=0, keepdims=True)
    out_p = p8 / (s8 + jnp.float32(1e-9))
    idx8 = 63 - (lax.bitcast_convert_type(keys8, jnp.int32) & 63)

    prob_ref[...] = out_p.T
    idx_ref[...] = idx8.T

    @pl.when(step == _GRID - 1)
    def _finish():
        scale = jnp.float32(float(_E) / (float(_T) * float(_T) * float(_K)))
        aux_ref[...] = scale * jnp.sum(psum_ref[...] * fsum_ref[...],
                                       axis=0, keepdims=True)


def kernel(x, W, b):
    b2 = b.reshape(_E, 1)
    idx, prob, aux = pl.pallas_call(
        _router_body,
        grid=(_GRID,),
        in_specs=[
            pl.BlockSpec((_E, _D), lambda i: (0, 0)),
            pl.BlockSpec((_BT, _D), lambda i: (i, 0)),
            pl.BlockSpec((_E, 1), lambda i: (0, 0)),
        ],
        out_specs=[
            pl.BlockSpec((_BT, _K), lambda i: (i, 0)),
            pl.BlockSpec((_BT, _K), lambda i: (i, 0)),
            pl.BlockSpec((1, 1), lambda i: (0, 0)),
        ],
        out_shape=[
            jax.ShapeDtypeStruct((_T, _K), jnp.int32),
            jax.ShapeDtypeStruct((_T, _K), jnp.float32),
            jax.ShapeDtypeStruct((1, 1), jnp.float32),
        ],
        scratch_shapes=[
            pltpu.VMEM((_E, 1), jnp.float32),
            pltpu.VMEM((_E, 1), jnp.float32),
        ],
        compiler_params=pltpu.CompilerParams(
            dimension_semantics=("arbitrary",),
        ),
    )(W, x, b2)
    return idx, prob, aux[0, 0]


# speedup vs baseline: 1.9616x; 1.0031x over previous
import jax
import jax.numpy as jnp
from jax import lax
from jax.experimental import pallas as pl
from jax.experimental.pallas import tpu as pltpu

_T = 16384
_D = 4096
_E = 64
_K = 8
_BT = 1024
_GRID = _T // _BT


def _router_body(w_ref, x_ref, b_ref, idx_ref, prob_ref, aux_ref,
                 psum_ref, fsum_ref):
    step = pl.program_id(0)

    @pl.when(step == 0)
    def _init():
        psum_ref[...] = jnp.zeros_like(psum_ref)
        fsum_ref[...] = jnp.zeros_like(fsum_ref)

    logits_t = lax.dot_general(
        w_ref[...], x_ref[...],
        dimension_numbers=(((1,), (1,)), ((), ())),
        preferred_element_type=jnp.float32)
    logits_t = logits_t + b_ref[...]

    row = lax.broadcasted_iota(jnp.int32, (_E, _BT), 0)
    neg_inf = jnp.float32(-jnp.inf)
    big = jnp.int32(_E)

    work = logits_t
    vals = []
    idxs = []
    for _ in range(_K):
        mx = jnp.max(work, axis=0, keepdims=True)
        hit0 = work == mx
        rsel = jnp.min(jnp.where(hit0, row, big), axis=0, keepdims=True)
        vals.append(mx)
        idxs.append(rsel)
        work = jnp.where(row == rsel, neg_inf, work)

    v8 = jnp.concatenate(vals, axis=0)
    i8 = jnp.concatenate(idxs, axis=0)

    m0 = vals[0]
    e_t = jnp.exp(logits_t - m0)
    zinv = jnp.float32(1.0) / jnp.sum(e_t, axis=0, keepdims=True)
    probs_t = e_t * zinv
    psum_ref[...] += jnp.sum(probs_t, axis=1, keepdims=True)

    sel = (work == neg_inf).astype(jnp.float32)
    fsum_ref[...] += jnp.sum(sel, axis=1, keepdims=True)

    p8 = jnp.exp(v8 - m0) * zinv
    s8 = jnp.sum(p8, axis=0, keepdims=True)
    out_p = p8 / (s8 + jnp.float32(1e-9))

    prob_ref[...] = out_p.T
    idx_ref[...] = i8.T

    @pl.when(step == _GRID - 1)
    def _finish():
        scale = jnp.float32(float(_E) / (float(_T) * float(_T) * float(_K)))
        aux_ref[...] = scale * jnp.sum(psum_ref[...] * fsum_ref[...],
                                       axis=0, keepdims=True)


def kernel(x, W, b):
    b2 = b.reshape(_E, 1)
    idx, prob, aux = pl.pallas_call(
        _router_body,
        grid=(_GRID,),
        in_specs=[
            pl.BlockSpec((_E, _D), lambda i: (0, 0)),
            pl.BlockSpec((_BT, _D), lambda i: (i, 0)),
            pl.BlockSpec((_E, 1), lambda i: (0, 0)),
        ],
        out_specs=[
            pl.BlockSpec((_BT, _K), lambda i: (i, 0)),
            pl.BlockSpec((_BT, _K), lambda i: (i, 0)),
            pl.BlockSpec((1, 1), lambda i: (0, 0)),
        ],
        out_shape=[
            jax.ShapeDtypeStruct((_T, _K), jnp.int32),
            jax.ShapeDtypeStruct((_T, _K), jnp.float32),
            jax.ShapeDtypeStruct((1, 1), jnp.float32),
        ],
        scratch_shapes=[
            pltpu.VMEM((_E, 1), jnp.float32),
            pltpu.VMEM((_E, 1), jnp.float32),
        ],
        compiler_params=pltpu.CompilerParams(
            dimension_semantics=("arbitrary",),
        ),
    )(W, x, b2)
    return idx, prob, aux[0, 0]
